# baseline (speedup 1.0000x reference)
import jax
import jax.numpy as jnp
from jax import lax
from jax.experimental import pallas as pl
from jax.experimental.pallas import tpu as pltpu
from jax.experimental.pallas import tpu_sc as plsc

ALPHA = 0.1
BETA = 0.5

NC = 2
NS = 16
NW = NC * NS
CHUNK = 128


def _prep_body(x_ref, h0_ref, sup_ref):
    sup_ref[...] = (1.0 - ALPHA) * x_ref[...] + ALPHA * h0_ref[...]


def _comb_body(p_ref, s_ref, w_ref, o_ref):
    dense = jnp.dot(s_ref[...], w_ref[...], preferred_element_type=jnp.float32)
    o_ref[...] = (1.0 - BETA) * (p_ref[0] + p_ref[1]) + BETA * dense


def _make_sc_agg(n, d, e):
    nb = e // CHUNK
    nb_w = nb // NW
    nb_r = nb % NW
    rfull = n // CHUNK
    rtail = n % CHUNK
    kmax = (rfull + (1 if rtail else 0) + NS - 1) // NS
    assert e % CHUNK == 0 and rtail % 8 == 0 and nb_w >= 2

    mesh = plsc.VectorSubcoreMesh(core_axis_name="c", subcore_axis_name="s")

    def body(sup_hbm, pk_hbm, out_hbm,
             pk_a, rows_a, pk_b, rows_b, idxr_sa, idxr_sb,
             agg_sh, sem_ia, sem_ib, sem_ga, sem_gb, sem_sa, sem_sb, sem_f):
        c = lax.axis_index("c")
        s = lax.axis_index("s")
        zero16 = jnp.zeros((16,), jnp.float32)

        def zfill(r, _):
            for b in range(d // 16):
                rows_a[r, pl.ds(b * 16, 16)] = zero16
            return 0
        lax.fori_loop(0, CHUNK, zfill, 0)
        for k in range(kmax):
            ci = k * NS + s

            @pl.when(ci < rfull)
            def _():
                pltpu.async_copy(rows_a, agg_sh.at[pl.ds(ci * CHUNK, CHUNK)], sem_f)

            if rtail:
                @pl.when(ci == rfull)
                def _():
                    pltpu.async_copy(rows_a.at[pl.ds(0, rtail)],
                                     agg_sh.at[pl.ds(rfull * CHUNK, rtail)], sem_f)
        for k in range(kmax):
            ci = k * NS + s

            @pl.when(ci < rfull)
            def _():
                pltpu.make_async_copy(
                    rows_a, agg_sh.at[pl.ds(0, CHUNK)], sem_f).wait()

            if rtail:
                @pl.when(ci == rfull)
                def _():
                    pltpu.make_async_copy(
                        rows_a.at[pl.ds(0, rtail)],
                        agg_sh.at[pl.ds(0, rtail)], sem_f).wait()
        plsc.subcore_barrier()

        w = c * NS + s
        ntask = nb_w + jnp.where(w < nb_r, 1, 0)
        cstart = w * nb_w + jnp.minimum(w, nb_r)
        clast = cstart + ntask - 1

        H = CHUNK // 2

        def scale_half(rv, pk, h):
            def group(g0, _):
                g = h * (H // 16) + g0
                w16 = lax.bitcast_convert_type(pk[2, pl.ds(g * 16, 16)],
                                               jnp.float32)
                for j in range(16):
                    wgt = w16[j]
                    i = g * 16 + j
                    for b in range(d // 16):
                        rv[i, pl.ds(b * 16, 16)] = rv[i, pl.ds(b * 16, 16)] * wgt
                return 0
            lax.fori_loop(0, H // 16, group, 0)

        def copy_ridx_half(pk, dst, h):
            for b in range(H // 16):
                dst[h, pl.ds(b * 16, 16)] = pk[1, pl.ds(h * H + b * 16, 16)]

        def process(rv, pk, sidx, s_s):
            scale_half(rv, pk, 0)
            copy_ridx_half(pk, sidx, 0)
            pltpu.async_copy(rv.at[pl.ds(0, H)], agg_sh.at[sidx.at[0]],
                             s_s, add=True)
            scale_half(rv, pk, 1)
            copy_ridx_half(pk, sidx, 1)
            pltpu.async_copy(rv.at[pl.ds(H, H)], agg_sh.at[sidx.at[1]],
                             s_s, add=True)

        def issue_idx(t, pk, s_i):
            pltpu.async_copy(pk_hbm.at[t], pk, s_i)

        def wait_idx(pk, s_i):
            pltpu.make_async_copy(pk_hbm.at[0], pk, s_i).wait()

        def wait_gather(rv, s_g):
            pltpu.make_async_copy(sup_hbm.at[pl.ds(0, CHUNK)], rv, s_g).wait()

        def wait_scatter(rv, sidx, s_s):
            pltpu.make_async_copy(rv.at[pl.ds(0, H)], agg_sh.at[sidx.at[0]],
                                  s_s).wait()
            pltpu.make_async_copy(rv.at[pl.ds(H, H)], agg_sh.at[sidx.at[1]],
                                  s_s).wait()

        def zfill_b(r, _):
            for b in range(d // 16):
                rows_b[r, pl.ds(b * 16, 16)] = zero16
            return 0
        lax.fori_loop(0, CHUNK, zfill_b, 0)
        for h in range(2):
            for b in range(H // 16):
                idxr_sb[h, pl.ds(b * 16, 16)] = jnp.zeros((16,), jnp.int32)
        pltpu.async_copy(rows_b.at[pl.ds(0, H)], agg_sh.at[idxr_sb.at[0]],
                         sem_sb, add=True)
        pltpu.async_copy(rows_b.at[pl.ds(H, H)], agg_sh.at[idxr_sb.at[1]],
                         sem_sb, add=True)
        issue_idx(cstart, pk_a, sem_ia)
        wait_idx(pk_a, sem_ia)
        pltpu.async_copy(sup_hbm.at[pk_a.at[0]], rows_a, sem_ga)
        issue_idx(cstart + 1, pk_b, sem_ib)

        def chunk_pair(u, _):
            nxa = jnp.minimum(cstart + 2 * u + 2, clast)
            nxb = jnp.minimum(cstart + 2 * u + 3, clast)
            wait_scatter(rows_b, idxr_sb, sem_sb)
            wait_idx(pk_b, sem_ib)
            pltpu.async_copy(sup_hbm.at[pk_b.at[0]], rows_b, sem_gb)
            wait_gather(rows_a, sem_ga)
            process(rows_a, pk_a, idxr_sa, sem_sa)
            issue_idx(nxa, pk_a, sem_ia)
            wait_gather(rows_b, sem_gb)
            process(rows_b, pk_b, idxr_sb, sem_sb)
            wait_scatter(rows_a, idxr_sa, sem_sa)
            wait_idx(pk_a, sem_ia)
            pltpu.async_copy(sup_hbm.at[pk_a.at[0]], rows_a, sem_ga)
            issue_idx(nxb, pk_b, sem_ib)
            return 0
        lax.fori_loop(0, ntask // 2, chunk_pair, 0)

        wait_scatter(rows_b, idxr_sb, sem_sb)
        wait_gather(rows_a, sem_ga)
        wait_idx(pk_b, sem_ib)

        @pl.when(ntask % 2 == 1)
        def _():
            process(rows_a, pk_a, idxr_sa, sem_sa)
            wait_scatter(rows_a, idxr_sa, sem_sa)

        plsc.subcore_barrier()

        for k in range(kmax):
            ci = k * NS + s

            @pl.when(ci < rfull)
            def _():
                pltpu.async_copy(agg_sh.at[pl.ds(ci * CHUNK, CHUNK)],
                                 out_hbm.at[c, pl.ds(ci * CHUNK, CHUNK)], sem_f)

            if rtail:
                @pl.when(ci == rfull)
                def _():
                    pltpu.async_copy(agg_sh.at[pl.ds(rfull * CHUNK, rtail)],
                                     out_hbm.at[c, pl.ds(rfull * CHUNK, rtail)],
                                     sem_f)
        for k in range(kmax):
            ci = k * NS + s

            @pl.when(ci < rfull)
            def _():
                pltpu.make_async_copy(agg_sh.at[pl.ds(0, CHUNK)],
                                      out_hbm.at[0, pl.ds(0, CHUNK)], sem_f).wait()

            if rtail:
                @pl.when(ci == rfull)
                def _():
                    pltpu.make_async_copy(agg_sh.at[pl.ds(0, rtail)],
                                          out_hbm.at[0, pl.ds(0, rtail)],
                                          sem_f).wait()

    return pl.kernel(
        body,
        out_type=jax.ShapeDtypeStruct((NC, n, d), jnp.float32),
        mesh=mesh,
        scratch_types=[
            pltpu.VMEM((3, CHUNK), jnp.int32),
            pltpu.VMEM((CHUNK, d), jnp.float32),
            pltpu.VMEM((3, CHUNK), jnp.int32),
            pltpu.VMEM((CHUNK, d), jnp.float32),
            pltpu.VMEM((2, CHUNK // 2), jnp.int32),
            pltpu.VMEM((2, CHUNK // 2), jnp.int32),
            pltpu.VMEM_SHARED((n, d), jnp.float32),
            pltpu.SemaphoreType.DMA,
            pltpu.SemaphoreType.DMA,
            pltpu.SemaphoreType.DMA,
            pltpu.SemaphoreType.DMA,
            pltpu.SemaphoreType.DMA,
            pltpu.SemaphoreType.DMA,
            pltpu.SemaphoreType.DMA,
        ],
    )


def kernel(x, edge_index, edge_weight, h0, W):
    n, d = x.shape
    e = edge_weight.shape[0]
    row = edge_index[0].astype(jnp.int32)
    col = edge_index[1].astype(jnp.int32)
    wbits = lax.bitcast_convert_type(edge_weight, jnp.int32)
    packed = (jnp.stack([col, row, wbits], axis=0)
              .reshape(3, e // CHUNK, CHUNK).transpose(1, 0, 2))

    rb = 1000
    grid = (n // rb,)
    support = pl.pallas_call(
        _prep_body,
        grid=grid,
        in_specs=[
            pl.BlockSpec((rb, d), lambda i: (i, 0)),
            pl.BlockSpec((rb, d), lambda i: (i, 0)),
        ],
        out_specs=pl.BlockSpec((rb, d), lambda i: (i, 0)),
        out_shape=jax.ShapeDtypeStruct((n, d), jnp.float32),
    )(x, h0)

    partial = _make_sc_agg(n, d, e)(support, packed)

    out = pl.pallas_call(
        _comb_body,
        grid=grid,
        in_specs=[
            pl.BlockSpec((NC, rb, d), lambda i: (0, i, 0)),
            pl.BlockSpec((rb, d), lambda i: (i, 0)),
            pl.BlockSpec((d, d), lambda i: (0, 0)),
        ],
        out_specs=pl.BlockSpec((rb, d), lambda i: (i, 0)),
        out_shape=jax.ShapeDtypeStruct((n, d), jnp.float32),
    )(partial, support, W)
    return out


# speedup vs baseline: 1.0164x; 1.0164x over previous
import jax
import jax.numpy as jnp
from jax import lax
from jax.experimental import pallas as pl
from jax.experimental.pallas import tpu as pltpu
from jax.experimental.pallas import tpu_sc as plsc

ALPHA = 0.1
BETA = 0.5

NC = 2
NS = 16
NW = NC * NS
CHUNK = 128


def _prep_body(x_ref, h0_ref, w_ref, sup_ref, dense_ref):
    sup = (1.0 - ALPHA) * x_ref[...] + ALPHA * h0_ref[...]
    sup_ref[...] = sup
    dense_ref[...] = jnp.dot(sup, w_ref[...], preferred_element_type=jnp.float32)


def _comb_body(p_ref, d_ref, o_ref):
    o_ref[...] = (1.0 - BETA) * (p_ref[0] + p_ref[1]) + BETA * d_ref[...]


def _make_sc_agg(n, d, e):
    nb = e // CHUNK
    nb_w = nb // NW
    nb_r = nb % NW
    rfull = n // CHUNK
    rtail = n % CHUNK
    kmax = (rfull + (1 if rtail else 0) + NS - 1) // NS
    assert e % CHUNK == 0 and rtail % 8 == 0 and nb_w >= 2

    mesh = plsc.VectorSubcoreMesh(core_axis_name="c", subcore_axis_name="s")

    def body(sup_hbm, pk_hbm, out_hbm,
             pk_a, rows_a, pk_b, rows_b, idxr_sa, idxr_sb,
             agg_sh, sem_ia, sem_ib, sem_ga, sem_gb, sem_sa, sem_sb, sem_f):
        c = lax.axis_index("c")
        s = lax.axis_index("s")
        zero16 = jnp.zeros((16,), jnp.float32)

        def zfill(r, _):
            for b in range(d // 16):
                rows_a[r, pl.ds(b * 16, 16)] = zero16
            return 0
        lax.fori_loop(0, CHUNK, zfill, 0)
        for k in range(kmax):
            ci = k * NS + s

            @pl.when(ci < rfull)
            def _():
                pltpu.async_copy(rows_a, agg_sh.at[pl.ds(ci * CHUNK, CHUNK)], sem_f)

            if rtail:
                @pl.when(ci == rfull)
                def _():
                    pltpu.async_copy(rows_a.at[pl.ds(0, rtail)],
                                     agg_sh.at[pl.ds(rfull * CHUNK, rtail)], sem_f)
        for k in range(kmax):
            ci = k * NS + s

            @pl.when(ci < rfull)
            def _():
                pltpu.make_async_copy(
                    rows_a, agg_sh.at[pl.ds(0, CHUNK)], sem_f).wait()

            if rtail:
                @pl.when(ci == rfull)
                def _():
                    pltpu.make_async_copy(
                        rows_a.at[pl.ds(0, rtail)],
                        agg_sh.at[pl.ds(0, rtail)], sem_f).wait()
        plsc.subcore_barrier()

        w = c * NS + s
        ntask = nb_w + jnp.where(w < nb_r, 1, 0)
        cstart = w * nb_w + jnp.minimum(w, nb_r)
        clast = cstart + ntask - 1

        def scale_rows(rv, pk):
            def group(g, _):
                w16 = lax.bitcast_convert_type(pk[2, pl.ds(g * 16, 16)],
                                               jnp.float32)
                for j in range(16):
                    wgt = w16[j]
                    i = g * 16 + j
                    for b in range(d // 16):
                        rv[i, pl.ds(b * 16, 16)] = rv[i, pl.ds(b * 16, 16)] * wgt
                return 0
            lax.fori_loop(0, CHUNK // 16, group, 0)

        def copy_ridx(pk, dst):
            for b in range(CHUNK // 16):
                dst[pl.ds(b * 16, 16)] = pk[1, pl.ds(b * 16, 16)]

        def issue_idx(t, pk, s_i):
            pltpu.async_copy(pk_hbm.at[t], pk, s_i)

        def wait_idx(pk, s_i):
            pltpu.make_async_copy(pk_hbm.at[0], pk, s_i).wait()

        def wait_gather(rv, s_g):
            pltpu.make_async_copy(sup_hbm.at[pl.ds(0, CHUNK)], rv, s_g).wait()

        def wait_scatter(rv, sidx, s_s):
            pltpu.make_async_copy(rv, agg_sh.at[sidx], s_s).wait()

        def zfill_b(r, _):
            for b in range(d // 16):
                rows_b[r, pl.ds(b * 16, 16)] = zero16
            return 0
        lax.fori_loop(0, CHUNK, zfill_b, 0)
        for b in range(CHUNK // 16):
            idxr_sb[pl.ds(b * 16, 16)] = jnp.zeros((16,), jnp.int32)
        pltpu.async_copy(rows_b, agg_sh.at[idxr_sb], sem_sb, add=True)
        issue_idx(cstart, pk_a, sem_ia)
        wait_idx(pk_a, sem_ia)
        pltpu.async_copy(sup_hbm.at[pk_a.at[0]], rows_a, sem_ga)
        issue_idx(cstart + 1, pk_b, sem_ib)

        def chunk_pair(u, _):
            nxa = jnp.minimum(cstart + 2 * u + 2, clast)
            nxb = jnp.minimum(cstart + 2 * u + 3, clast)
            wait_scatter(rows_b, idxr_sb, sem_sb)
            wait_idx(pk_b, sem_ib)
            pltpu.async_copy(sup_hbm.at[pk_b.at[0]], rows_b, sem_gb)
            wait_gather(rows_a, sem_ga)
            scale_rows(rows_a, pk_a)
            copy_ridx(pk_a, idxr_sa)
            pltpu.async_copy(rows_a, agg_sh.at[idxr_sa], sem_sa, add=True)
            issue_idx(nxa, pk_a, sem_ia)
            wait_gather(rows_b, sem_gb)
            scale_rows(rows_b, pk_b)
            wait_scatter(rows_a, idxr_sa, sem_sa)
            wait_idx(pk_a, sem_ia)
            pltpu.async_copy(sup_hbm.at[pk_a.at[0]], rows_a, sem_ga)
            copy_ridx(pk_b, idxr_sb)
            pltpu.async_copy(rows_b, agg_sh.at[idxr_sb], sem_sb, add=True)
            issue_idx(nxb, pk_b, sem_ib)
            return 0
        lax.fori_loop(0, ntask // 2, chunk_pair, 0)

        wait_scatter(rows_b, idxr_sb, sem_sb)
        wait_gather(rows_a, sem_ga)
        wait_idx(pk_b, sem_ib)

        @pl.when(ntask % 2 == 1)
        def _():
            scale_rows(rows_a, pk_a)
            copy_ridx(pk_a, idxr_sa)
            pltpu.sync_copy(rows_a, agg_sh.at[idxr_sa], add=True)

        plsc.subcore_barrier()

        for k in range(kmax):
            ci = k * NS + s

            @pl.when(ci < rfull)
            def _():
                pltpu.async_copy(agg_sh.at[pl.ds(ci * CHUNK, CHUNK)],
                                 out_hbm.at[c, pl.ds(ci * CHUNK, CHUNK)], sem_f)

            if rtail:
                @pl.when(ci == rfull)
                def _():
                    pltpu.async_copy(agg_sh.at[pl.ds(rfull * CHUNK, rtail)],
                                     out_hbm.at[c, pl.ds(rfull * CHUNK, rtail)],
                                     sem_f)
        for k in range(kmax):
            ci = k * NS + s

            @pl.when(ci < rfull)
            def _():
                pltpu.make_async_copy(agg_sh.at[pl.ds(0, CHUNK)],
                                      out_hbm.at[0, pl.ds(0, CHUNK)], sem_f).wait()

            if rtail:
                @pl.when(ci == rfull)
                def _():
                    pltpu.make_async_copy(agg_sh.at[pl.ds(0, rtail)],
                                          out_hbm.at[0, pl.ds(0, rtail)],
                                          sem_f).wait()

    return pl.kernel(
        body,
        out_type=jax.ShapeDtypeStruct((NC, n, d), jnp.float32),
        mesh=mesh,
        scratch_types=[
            pltpu.VMEM((3, CHUNK), jnp.int32),
            pltpu.VMEM((CHUNK, d), jnp.float32),
            pltpu.VMEM((3, CHUNK), jnp.int32),
            pltpu.VMEM((CHUNK, d), jnp.float32),
            pltpu.VMEM((CHUNK,), jnp.int32),
            pltpu.VMEM((CHUNK,), jnp.int32),
            pltpu.VMEM_SHARED((n, d), jnp.float32),
            pltpu.SemaphoreType.DMA,
            pltpu.SemaphoreType.DMA,
            pltpu.SemaphoreType.DMA,
            pltpu.SemaphoreType.DMA,
            pltpu.SemaphoreType.DMA,
            pltpu.SemaphoreType.DMA,
            pltpu.SemaphoreType.DMA,
        ],
    )


def kernel(x, edge_index, edge_weight, h0, W):
    n, d = x.shape
    e = edge_weight.shape[0]
    row = edge_index[0].astype(jnp.int32)
    col = edge_index[1].astype(jnp.int32)
    wbits = lax.bitcast_convert_type(edge_weight, jnp.int32)
    packed = (jnp.stack([col, row, wbits], axis=0)
              .reshape(3, e // CHUNK, CHUNK).transpose(1, 0, 2))

    rb = 1000
    grid = (n // rb,)
    support, dense = pl.pallas_call(
        _prep_body,
        grid=grid,
        in_specs=[
            pl.BlockSpec((rb, d), lambda i: (i, 0)),
            pl.BlockSpec((rb, d), lambda i: (i, 0)),
            pl.BlockSpec((d, d), lambda i: (0, 0)),
        ],
        out_specs=[
            pl.BlockSpec((rb, d), lambda i: (i, 0)),
            pl.BlockSpec((rb, d), lambda i: (i, 0)),
        ],
        out_shape=[
            jax.ShapeDtypeStruct((n, d), jnp.float32),
            jax.ShapeDtypeStruct((n, d), jnp.float32),
        ],
    )(x, h0, W)

    partial = _make_sc_agg(n, d, e)(support, packed)

    out = pl.pallas_call(
        _comb_body,
        grid=grid,
        in_specs=[
            pl.BlockSpec((NC, rb, d), lambda i: (0, i, 0)),
            pl.BlockSpec((rb, d), lambda i: (i, 0)),
        ],
        out_specs=pl.BlockSpec((rb, d), lambda i: (i, 0)),
        out_shape=jax.ShapeDtypeStruct((n, d), jnp.float32),
    )(partial, dense)
    return out
